# baseline (speedup 1.0000x reference)
import functools

import jax
import jax.numpy as jnp
import numpy as np
from jax.experimental import pallas as pl
from jax.experimental.pallas import tpu as pltpu

B = 16384
D = 256
T = 26
P = 128
K = T * P
COMMIT = 0.25
TEMP = 0.07

ROWS = 256
NT = B // ROWS + T
BP = NT * ROWS


def _main_body(tt_ref, cnt_ref, x_ref, emb_ref, qst_ref, enc_ref, sq_ref):
    i = pl.program_id(0)
    t = tt_ref[i]
    cnt = cnt_ref[i]
    xt = x_ref[...]
    embt = emb_ref[...]

    norm = jnp.sqrt(jnp.sum(xt * xt, axis=1, keepdims=True))
    xn = xt / jnp.maximum(norm, 1e-12)

    xsq = jnp.sum(xn * xn, axis=1, keepdims=True)
    esq = jnp.sum(embt * embt, axis=1)

    s = jax.lax.dot_general(xn, embt, (((1,), (1,)), ((), ())),
                            preferred_element_type=jnp.float32)
    d = xsq + esq[None, :] - 2.0 * s

    li = jnp.argmin(d, axis=1).astype(jnp.int32)
    enc = t * P + li

    ohp = (li[:, None] == jax.lax.broadcasted_iota(jnp.int32, (ROWS, P), 1))
    ohp = ohp.astype(jnp.float32)
    qrow = jax.lax.dot_general(ohp, embt, (((1,), (0,)), ((), ())),
                               preferred_element_type=jnp.float32)
    qn = jnp.sqrt(jnp.sum(qrow * qrow, axis=1, keepdims=True))
    qrow = qrow / jnp.maximum(qn, 1e-12)

    qst_ref[...] = xn + (qrow - xn)
    enc_ref[0, 0, :] = enc

    rvalid = (jax.lax.broadcasted_iota(jnp.int32, (ROWS, 1), 0) < cnt)
    diff = jnp.where(rvalid, qrow - xn, 0.0)
    part = jnp.sum(diff * diff).reshape(1, 1)

    @pl.when(i == 0)
    def _():
        sq_ref[...] = jnp.zeros((1, 1), jnp.float32)

    sq_ref[...] += part


def _uniform_body(emb_ref, sel_ref, lab_ref, noteye_ref, valid_ref, out_ref):
    emb = emb_ref[...]
    nrm = jnp.sqrt(jnp.sum(emb * emb, axis=1, keepdims=True))
    nemb = emb / jnp.maximum(nrm, 1e-12)
    se = jax.lax.dot_general(sel_ref[...], nemb, (((1,), (0,)), ((), ())),
                             preferred_element_type=jnp.float32)
    sim = jax.lax.dot_general(se, se, (((1,), (1,)), ((), ())),
                              preferred_element_type=jnp.float32)
    e = jnp.exp(sim / TEMP) * noteye_ref[...]
    sum_exp = jnp.sum(e, axis=1, keepdims=True)
    pos_sum = jnp.sum(e * lab_ref[...], axis=1, keepdims=True)
    valid = valid_ref[...]
    term = jnp.where(valid > 0.0,
                     jnp.log(pos_sum / jnp.maximum(sum_exp, 1e-30) + 1e-45),
                     0.0)
    n_valid = jnp.sum(valid)
    out_ref[...] = (-jnp.sum(term * valid) / n_valid).reshape(1, 1)


def _uniform_loss(embeddings):
    sampled_num = int(0.1 * P)
    perm = jax.random.permutation(jax.random.key(42), P)[:sampled_num]
    all_idx = jnp.arange(K).reshape(T, P)
    sampled_indices = all_idx[:, perm].reshape(-1)
    S = T * sampled_num
    SP = 384
    sel = (sampled_indices[:, None] ==
           jnp.arange(K)[None, :]).astype(jnp.float32)
    sel = jnp.pad(sel, ((0, SP - S), (0, 0)))
    labels = sampled_indices // P
    lab = (labels[None, :] == labels[:, None]).astype(jnp.float32)
    lab = jnp.pad(lab, ((0, SP - S), (0, SP - S)))
    noteye = 1.0 - jnp.eye(SP, dtype=jnp.float32)
    colvalid = jnp.pad(jnp.ones((S,), jnp.float32), (0, SP - S))
    noteye = noteye * colvalid[None, :] * colvalid[:, None]
    valid = colvalid[:, None]
    uni = pl.pallas_call(
        _uniform_body,
        out_shape=jax.ShapeDtypeStruct((1, 1), jnp.float32),
    )(embeddings, sel, lab, noteye, valid)
    return uni[0, 0]


@jax.jit
def kernel(x, Q, embeddings):
    oh = (Q[:, None] == jnp.arange(T)[None, :]).astype(jnp.int32)
    cum = jnp.cumsum(oh, axis=0)
    counts = cum[-1]
    rank = jnp.take_along_axis(cum, Q[:, None], axis=1)[:, 0] - 1
    tiles_per = (counts + ROWS - 1) // ROWS
    pad_off = jnp.concatenate([jnp.zeros((1,), jnp.int32),
                               jnp.cumsum(tiles_per * ROWS)[:-1]])
    pos = pad_off[Q] + rank
    perm_pad = jnp.zeros((BP,), jnp.int32).at[pos].set(
        jnp.arange(B, dtype=jnp.int32))
    tile_start = jnp.concatenate([jnp.zeros((1,), jnp.int32),
                                  jnp.cumsum(tiles_per)[:-1]])
    tt = jnp.repeat(jnp.arange(T, dtype=jnp.int32), tiles_per,
                    total_repeat_length=NT)
    k_within = jnp.arange(NT, dtype=jnp.int32) - tile_start[tt]
    cnt = jnp.clip(counts[tt] - k_within * ROWS, 0, ROWS).astype(jnp.int32)

    xs = x[perm_pad]

    grid_spec = pltpu.PrefetchScalarGridSpec(
        num_scalar_prefetch=2,
        grid=(NT,),
        in_specs=[
            pl.BlockSpec((ROWS, D), lambda i, tt_r, cnt_r: (i, 0)),
            pl.BlockSpec((P, D), lambda i, tt_r, cnt_r: (tt_r[i], 0)),
        ],
        out_specs=[
            pl.BlockSpec((ROWS, D), lambda i, tt_r, cnt_r: (i, 0)),
            pl.BlockSpec((1, 1, ROWS), lambda i, tt_r, cnt_r: (i, 0, 0)),
            pl.BlockSpec((1, 1), lambda i, tt_r, cnt_r: (0, 0)),
        ],
    )
    qst_s, enc_s, sqsum = pl.pallas_call(
        _main_body,
        grid_spec=grid_spec,
        out_shape=[
            jax.ShapeDtypeStruct((BP, D), jnp.float32),
            jax.ShapeDtypeStruct((NT, 1, ROWS), jnp.int32),
            jax.ShapeDtypeStruct((1, 1), jnp.float32),
        ],
    )(tt, cnt, xs, embeddings)

    loss = (1.0 + COMMIT) * (sqsum[0, 0] / (B * D))

    qst = qst_s[pos]
    enc = enc_s.reshape(BP)[pos]

    return (qst, loss, _uniform_loss(embeddings), enc)
